# baseline (speedup 1.0000x reference)
import functools

import jax
import jax.numpy as jnp
from jax import lax
from jax.experimental import pallas as pl
from jax.experimental.pallas import tpu as pltpu
from jax.experimental.pallas import tpu_sc as plsc

_N = 10000
_E = 320000
_D = 128
_C = 40

_NP = 10240
_RT = _NP // 16
_DUMMY = _N

_K = 128
_CH0 = 32
_CH1 = 128
_BC = 16
_NBLK = 8
_EP = 16 * (_CH0 + _CH1) * _K

_BLK = 512


_sc_mesh = plsc.VectorSubcoreMesh(core_axis_name="c", subcore_axis_name="s")


@functools.partial(
    pl.kernel,
    mesh=_sc_mesh,
    out_type=jax.ShapeDtypeStruct((2, _NP, _D), jnp.float32),
    scratch_types=[
        pltpu.VMEM((2, _BC, _K), jnp.int32),
        pltpu.VMEM((2, _BC, _K), jnp.int32),
        pltpu.VMEM((_K, _D), jnp.float32),
        pltpu.VMEM((_K, _D), jnp.float32),
        pltpu.VMEM_SHARED((_NP, _D), jnp.float32),
        pltpu.SemaphoreType.DMA,
        pltpu.SemaphoreType.DMA,
        pltpu.SemaphoreType.DMA,
        pltpu.SemaphoreType.DMA,
    ],
)
def _sc_propagate(y_hbm, src_hbm, dst_hbm, zeros_hbm, out_hbm,
                  src_b, dst_b, rows0, rows1, acc, sem0, sem1, isem, zsem):
    cid = lax.axis_index("c")
    sid = lax.axis_index("s")
    row0 = sid * _RT
    nch = lax.select(cid == 0, _CH0, _CH1)

    pltpu.async_copy(src_hbm.at[cid, sid, 0], src_b.at[0], isem)
    pltpu.async_copy(dst_hbm.at[cid, sid, 0], dst_b.at[0], isem)
    pltpu.async_copy(src_hbm.at[cid, sid, 1], src_b.at[1], isem)
    pltpu.async_copy(dst_hbm.at[cid, sid, 1], dst_b.at[1], isem)

    def zbody(i, carry):
        pltpu.async_copy(zeros_hbm, acc.at[pl.ds(row0 + 40 * i, 40)], zsem)
        return carry

    lax.fori_loop(0, _RT // 40, zbody, 0)

    for _ in range(2):
        pltpu.make_async_copy(src_hbm.at[cid, sid, 0], src_b.at[0],
                              isem).wait()
        pltpu.make_async_copy(dst_hbm.at[cid, sid, 0], dst_b.at[0],
                              isem).wait()

    pltpu.async_copy(y_hbm.at[src_b.at[0, 0]], rows0, sem0)
    pltpu.async_copy(y_hbm.at[src_b.at[0, 1]], rows1, sem1)

    def zdrain(i, carry):
        pltpu.make_async_copy(zeros_hbm, acc.at[pl.ds(row0, 40)],
                              zsem).wait()
        return carry

    lax.fori_loop(0, _RT // 40, zdrain, 0)
    plsc.subcore_barrier()

    def body(g, carry):
        c0 = 2 * g

        @pl.when((c0 % _BC == 0) & (c0 > 0) & (c0 < nch - _BC))
        def _():
            k1 = c0 // _BC + 1
            pltpu.async_copy(src_hbm.at[cid, sid, k1],
                             src_b.at[k1 % 2], isem)
            pltpu.async_copy(dst_hbm.at[cid, sid, k1],
                             dst_b.at[k1 % 2], isem)

        @pl.when(((c0 + 2) % _BC == 0) & (c0 + 2 >= 2 * _BC)
                 & (c0 + 2 < nch))
        def _():
            pltpu.make_async_copy(src_hbm.at[cid, sid, 0], src_b.at[0],
                                  isem).wait()
            pltpu.make_async_copy(dst_hbm.at[cid, sid, 0], dst_b.at[0],
                                  isem).wait()

        par = (c0 // _BC) % 2
        ci = c0 % _BC

        pltpu.make_async_copy(y_hbm.at[src_b.at[0, 0]], rows0, sem0).wait()
        pltpu.sync_copy(rows0, acc.at[dst_b.at[par, ci]], add=True)

        @pl.when(c0 + 2 < nch)
        def _():
            cn = c0 + 2
            pltpu.async_copy(y_hbm.at[src_b.at[(cn // _BC) % 2, cn % _BC]],
                             rows0, sem0)

        pltpu.make_async_copy(y_hbm.at[src_b.at[0, 0]], rows1, sem1).wait()
        pltpu.sync_copy(rows1, acc.at[dst_b.at[par, ci + 1]], add=True)

        @pl.when(c0 + 3 < nch)
        def _():
            cn = c0 + 3
            pltpu.async_copy(y_hbm.at[src_b.at[(cn // _BC) % 2, cn % _BC]],
                             rows1, sem1)

        return carry

    lax.fori_loop(0, nch // 2, body, 0)
    plsc.subcore_barrier()
    pltpu.sync_copy(acc.at[pl.ds(row0, _RT)],
                    out_hbm.at[cid, pl.ds(row0, _RT)])


def _mm_first_body(x_ref, w_ref, o_ref):
    o_ref[...] = lax.dot_general(
        x_ref[...], w_ref[...], (((1,), (0,)), ((), ())),
        precision=lax.Precision.HIGHEST, preferred_element_type=jnp.float32)


def _mm_mid_body(p_ref, b_ref, w_ref, o_ref):
    h = jnp.maximum(p_ref[0] + p_ref[1] + b_ref[...], 0.0)
    o_ref[...] = lax.dot_general(
        h, w_ref[...], (((1,), (0,)), ((), ())),
        precision=lax.Precision.HIGHEST, preferred_element_type=jnp.float32)


def _relu_body(p_ref, b_ref, o_ref):
    o_ref[...] = jnp.maximum(p_ref[0] + p_ref[1] + b_ref[...], 0.0)


def _mm_first(x, w):
    return pl.pallas_call(
        _mm_first_body,
        grid=(_NP // _BLK,),
        in_specs=[
            pl.BlockSpec((_BLK, _D), lambda i: (i, 0)),
            pl.BlockSpec((_D, _D), lambda i: (0, 0)),
        ],
        out_specs=pl.BlockSpec((_BLK, _D), lambda i: (i, 0)),
        out_shape=jax.ShapeDtypeStruct((_NP, _D), jnp.float32),
    )(x, w)


def _mm_mid(p, b, w):
    return pl.pallas_call(
        _mm_mid_body,
        grid=(_NP // _BLK,),
        in_specs=[
            pl.BlockSpec((2, _BLK, _D), lambda i: (0, i, 0)),
            pl.BlockSpec((1, _D), lambda i: (0, 0)),
            pl.BlockSpec((_D, _D), lambda i: (0, 0)),
        ],
        out_specs=pl.BlockSpec((_BLK, _D), lambda i: (i, 0)),
        out_shape=jax.ShapeDtypeStruct((_NP, _D), jnp.float32),
    )(p, b.reshape(1, _D), w)


def _relu_out(p, b):
    return pl.pallas_call(
        _relu_body,
        grid=(_NP // _BLK,),
        in_specs=[
            pl.BlockSpec((2, _BLK, _D), lambda i: (0, i, 0)),
            pl.BlockSpec((1, _D), lambda i: (0, 0)),
        ],
        out_specs=pl.BlockSpec((_BLK, _D), lambda i: (i, 0)),
        out_shape=jax.ShapeDtypeStruct((_NP, _D), jnp.float32),
    )(p, b.reshape(1, _D))


def kernel(features, edge_index, W1, b1, W2, b2, W3, b3):
    f = jnp.pad(features, ((0, _NP - _N), (0, 0)))
    srcp = jnp.pad(edge_index[0], (0, _EP - _E))
    dstp = jnp.pad(edge_index[1], (0, _EP - _E), constant_values=_DUMMY)

    def _per_core(flat, fill):
        per_tile = flat.reshape(16, _CH0 + _CH1, _K)
        c0 = per_tile[:, :_CH0].reshape(16, _CH0 // _BC, _BC, _K)
        c1 = per_tile[:, _CH0:].reshape(16, _CH1 // _BC, _BC, _K)
        c0 = jnp.pad(c0, ((0, 0), (0, _NBLK - _CH0 // _BC), (0, 0), (0, 0)),
                     constant_values=fill)
        c1 = jnp.pad(c1, ((0, 0), (0, _NBLK - _CH1 // _BC), (0, 0), (0, 0)),
                     constant_values=fill)
        return jnp.stack([c0, c1])

    src = _per_core(srcp, 0)
    dst = _per_core(dstp, _DUMMY)
    w3p = jnp.pad(W3, ((0, 0), (0, _D - _C)))
    b3p = jnp.pad(b3, (0, _D - _C))
    z = jnp.zeros((40, _D), jnp.float32)

    y1 = _mm_first(f, W1)
    p1 = _sc_propagate(y1, src, dst, z)
    y2 = _mm_mid(p1, b1, W2)
    p2 = _sc_propagate(y2, src, dst, z)
    y3 = _mm_mid(p2, b2, w3p)
    p3 = _sc_propagate(y3, src, dst, z)
    out = _relu_out(p3, b3p)
    return out[:_N, :_C]


# speedup vs baseline: 1.0628x; 1.0628x over previous
import functools

import jax
import jax.numpy as jnp
from jax import lax
from jax.experimental import pallas as pl
from jax.experimental.pallas import tpu as pltpu
from jax.experimental.pallas import tpu_sc as plsc

_N = 10000
_E = 320000
_D = 128
_C = 40

_NP = 10240
_RT = _NP // 16
_DUMMY = _N

_K = 128
_CH = 80
_BC = 16
_NBLK = _CH // _BC
_EP = 2 * 16 * _CH * _K

_BLK = 512


_sc_mesh = plsc.VectorSubcoreMesh(core_axis_name="c", subcore_axis_name="s")


@functools.partial(
    pl.kernel,
    mesh=_sc_mesh,
    out_type=jax.ShapeDtypeStruct((2, _NP, _D), jnp.float32),
    scratch_types=[
        pltpu.VMEM((2, _BC, _K), jnp.int32),
        pltpu.VMEM((2, _BC, _K), jnp.int32),
        pltpu.VMEM((_K, _D), jnp.float32),
        pltpu.VMEM((_K, _D), jnp.float32),
        pltpu.VMEM_SHARED((_NP, _D), jnp.float32),
        pltpu.SemaphoreType.DMA,
        pltpu.SemaphoreType.DMA,
        pltpu.SemaphoreType.DMA,
        pltpu.SemaphoreType.DMA,
    ],
)
def _sc_propagate(y_hbm, src_hbm, dst_hbm, zeros_hbm, out_hbm,
                  src_b, dst_b, rows0, rows1, acc, sem0, sem1, isem, zsem):
    cid = lax.axis_index("c")
    sid = lax.axis_index("s")
    row0 = sid * _RT

    pltpu.async_copy(src_hbm.at[cid, sid, 0], src_b.at[0], isem)
    pltpu.async_copy(dst_hbm.at[cid, sid, 0], dst_b.at[0], isem)
    pltpu.async_copy(src_hbm.at[cid, sid, 1], src_b.at[1], isem)
    pltpu.async_copy(dst_hbm.at[cid, sid, 1], dst_b.at[1], isem)

    def zbody(i, carry):
        pltpu.async_copy(zeros_hbm, acc.at[pl.ds(row0 + 40 * i, 40)], zsem)
        return carry

    lax.fori_loop(0, _RT // 40, zbody, 0)

    for _ in range(2):
        pltpu.make_async_copy(src_hbm.at[cid, sid, 0], src_b.at[0],
                              isem).wait()
        pltpu.make_async_copy(dst_hbm.at[cid, sid, 0], dst_b.at[0],
                              isem).wait()

    pltpu.async_copy(y_hbm.at[src_b.at[0, 0]], rows0, sem0)
    pltpu.async_copy(y_hbm.at[src_b.at[0, 1]], rows1, sem1)

    def zdrain(i, carry):
        pltpu.make_async_copy(zeros_hbm, acc.at[pl.ds(row0, 40)],
                              zsem).wait()
        return carry

    lax.fori_loop(0, _RT // 40, zdrain, 0)
    plsc.subcore_barrier()

    def body(g, carry):
        c0 = 2 * g

        @pl.when((c0 % _BC == 0) & (c0 > 0) & (c0 < (_NBLK - 1) * _BC))
        def _():
            k1 = c0 // _BC + 1
            pltpu.async_copy(src_hbm.at[cid, sid, k1],
                             src_b.at[k1 % 2], isem)
            pltpu.async_copy(dst_hbm.at[cid, sid, k1],
                             dst_b.at[k1 % 2], isem)

        @pl.when(((c0 + 2) % _BC == 0) & (c0 + 2 >= 2 * _BC)
                 & (c0 + 2 < _CH))
        def _():
            pltpu.make_async_copy(src_hbm.at[cid, sid, 0], src_b.at[0],
                                  isem).wait()
            pltpu.make_async_copy(dst_hbm.at[cid, sid, 0], dst_b.at[0],
                                  isem).wait()

        par = (c0 // _BC) % 2
        ci = c0 % _BC

        pltpu.make_async_copy(y_hbm.at[src_b.at[0, 0]], rows0, sem0).wait()
        pltpu.sync_copy(rows0, acc.at[dst_b.at[par, ci]], add=True)

        @pl.when(c0 + 2 < _CH)
        def _():
            cn = c0 + 2
            pltpu.async_copy(y_hbm.at[src_b.at[(cn // _BC) % 2, cn % _BC]],
                             rows0, sem0)

        pltpu.make_async_copy(y_hbm.at[src_b.at[0, 0]], rows1, sem1).wait()
        pltpu.sync_copy(rows1, acc.at[dst_b.at[par, ci + 1]], add=True)

        @pl.when(c0 + 3 < _CH)
        def _():
            cn = c0 + 3
            pltpu.async_copy(y_hbm.at[src_b.at[(cn // _BC) % 2, cn % _BC]],
                             rows1, sem1)

        return carry

    lax.fori_loop(0, _CH // 2, body, 0)
    plsc.subcore_barrier()
    pltpu.sync_copy(acc.at[pl.ds(row0, _RT)],
                    out_hbm.at[cid, pl.ds(row0, _RT)])


def _mm_first_body(x_ref, w_ref, o_ref):
    o_ref[...] = lax.dot_general(
        x_ref[...], w_ref[...], (((1,), (0,)), ((), ())),
        precision=lax.Precision.HIGHEST, preferred_element_type=jnp.float32)


def _mm_mid_body(p_ref, b_ref, w_ref, o_ref):
    h = jnp.maximum(p_ref[0] + p_ref[1] + b_ref[...], 0.0)
    o_ref[...] = lax.dot_general(
        h, w_ref[...], (((1,), (0,)), ((), ())),
        precision=lax.Precision.HIGHEST, preferred_element_type=jnp.float32)


def _relu_body(p_ref, b_ref, o_ref):
    o_ref[...] = jnp.maximum(p_ref[0] + p_ref[1] + b_ref[...], 0.0)


def _mm_first(x, w):
    return pl.pallas_call(
        _mm_first_body,
        grid=(_NP // _BLK,),
        in_specs=[
            pl.BlockSpec((_BLK, _D), lambda i: (i, 0)),
            pl.BlockSpec((_D, _D), lambda i: (0, 0)),
        ],
        out_specs=pl.BlockSpec((_BLK, _D), lambda i: (i, 0)),
        out_shape=jax.ShapeDtypeStruct((_NP, _D), jnp.float32),
    )(x, w)


def _mm_mid(p, b, w):
    return pl.pallas_call(
        _mm_mid_body,
        grid=(_NP // _BLK,),
        in_specs=[
            pl.BlockSpec((2, _BLK, _D), lambda i: (0, i, 0)),
            pl.BlockSpec((1, _D), lambda i: (0, 0)),
            pl.BlockSpec((_D, _D), lambda i: (0, 0)),
        ],
        out_specs=pl.BlockSpec((_BLK, _D), lambda i: (i, 0)),
        out_shape=jax.ShapeDtypeStruct((_NP, _D), jnp.float32),
    )(p, b.reshape(1, _D), w)


def _relu_out(p, b):
    return pl.pallas_call(
        _relu_body,
        grid=(_NP // _BLK,),
        in_specs=[
            pl.BlockSpec((2, _BLK, _D), lambda i: (0, i, 0)),
            pl.BlockSpec((1, _D), lambda i: (0, 0)),
        ],
        out_specs=pl.BlockSpec((_BLK, _D), lambda i: (i, 0)),
        out_shape=jax.ShapeDtypeStruct((_NP, _D), jnp.float32),
    )(p, b.reshape(1, _D))


def kernel(features, edge_index, W1, b1, W2, b2, W3, b3):
    f = jnp.pad(features, ((0, _NP - _N), (0, 0)))
    src = jnp.pad(edge_index[0],
                  (0, _EP - _E)).reshape(2, 16, _NBLK, _BC, _K)
    dst = jnp.pad(edge_index[1], (0, _EP - _E),
                  constant_values=_DUMMY).reshape(2, 16, _NBLK, _BC, _K)
    w3p = jnp.pad(W3, ((0, 0), (0, _D - _C)))
    b3p = jnp.pad(b3, (0, _D - _C))
    z = jnp.zeros((40, _D), jnp.float32)

    y1 = _mm_first(f, W1)
    p1 = _sc_propagate(y1, src, dst, z)
    y2 = _mm_mid(p1, b1, W2)
    p2 = _sc_propagate(y2, src, dst, z)
    y3 = _mm_mid(p2, b2, w3p)
    p3 = _sc_propagate(y3, src, dst, z)
    out = _relu_out(p3, b3p)
    return out[:_N, :_C]


# speedup vs baseline: 1.1436x; 1.0760x over previous
import functools

import jax
import jax.numpy as jnp
from jax import lax
from jax.experimental import pallas as pl
from jax.experimental.pallas import tpu as pltpu
from jax.experimental.pallas import tpu_sc as plsc

_N = 10000
_E = 320000
_D = 128
_C = 40

_NP = 10240
_RT = _NP // 16
_DUMMY = _N

_K = 128
_CH = 80
_BC = 16
_NBLK = _CH // _BC
_EP = 2 * 16 * _CH * _K

_BLK = 512


_sc_mesh = plsc.VectorSubcoreMesh(core_axis_name="c", subcore_axis_name="s")


@functools.partial(
    pl.kernel,
    mesh=_sc_mesh,
    out_type=jax.ShapeDtypeStruct((2, _NP, _D), jnp.float32),
    scratch_types=[
        pltpu.VMEM((2, _BC, _K), jnp.int32),
        pltpu.VMEM((2, _BC, _K), jnp.int32),
        pltpu.VMEM((_K, _D), jnp.float32),
        pltpu.VMEM((_K, _D), jnp.float32),
        pltpu.VMEM((40, _D), jnp.float32),
        pltpu.VMEM_SHARED((_NP, _D), jnp.float32),
        pltpu.SemaphoreType.DMA,
        pltpu.SemaphoreType.DMA,
        pltpu.SemaphoreType.DMA,
        pltpu.SemaphoreType.DMA,
    ],
)
def _sc_propagate(y_hbm, src_hbm, dst_hbm, zeros_hbm, out_hbm,
                  src_b, dst_b, rows0, rows1, zbuf, acc,
                  sem0, sem1, isem, zsem):
    cid = lax.axis_index("c")
    sid = lax.axis_index("s")
    row0 = sid * _RT

    pltpu.async_copy(src_hbm.at[cid, sid, 0], src_b.at[0], isem)
    pltpu.async_copy(dst_hbm.at[cid, sid, 0], dst_b.at[0], isem)
    pltpu.async_copy(src_hbm.at[cid, sid, 1], src_b.at[1], isem)
    pltpu.async_copy(dst_hbm.at[cid, sid, 1], dst_b.at[1], isem)

    pltpu.sync_copy(zeros_hbm, zbuf)

    def zbody(i, carry):
        pltpu.async_copy(zbuf, acc.at[pl.ds(row0 + 40 * i, 40)], zsem)
        return carry

    lax.fori_loop(0, _RT // 40, zbody, 0)

    for _ in range(2):
        pltpu.make_async_copy(src_hbm.at[cid, sid, 0], src_b.at[0],
                              isem).wait()
        pltpu.make_async_copy(dst_hbm.at[cid, sid, 0], dst_b.at[0],
                              isem).wait()

    pltpu.async_copy(y_hbm.at[src_b.at[0, 0]], rows0, sem0)
    pltpu.async_copy(y_hbm.at[src_b.at[0, 1]], rows1, sem1)

    def zdrain(i, carry):
        pltpu.make_async_copy(zbuf, acc.at[pl.ds(row0, 40)],
                              zsem).wait()
        return carry

    lax.fori_loop(0, _RT // 40, zdrain, 0)
    plsc.subcore_barrier()

    def body(g, carry):
        c0 = 2 * g

        @pl.when((c0 % _BC == 0) & (c0 > 0) & (c0 < (_NBLK - 1) * _BC))
        def _():
            k1 = c0 // _BC + 1
            pltpu.async_copy(src_hbm.at[cid, sid, k1],
                             src_b.at[k1 % 2], isem)
            pltpu.async_copy(dst_hbm.at[cid, sid, k1],
                             dst_b.at[k1 % 2], isem)

        @pl.when(((c0 + 2) % _BC == 0) & (c0 + 2 >= 2 * _BC)
                 & (c0 + 2 < _CH))
        def _():
            pltpu.make_async_copy(src_hbm.at[cid, sid, 0], src_b.at[0],
                                  isem).wait()
            pltpu.make_async_copy(dst_hbm.at[cid, sid, 0], dst_b.at[0],
                                  isem).wait()

        par = (c0 // _BC) % 2
        ci = c0 % _BC

        pltpu.make_async_copy(y_hbm.at[src_b.at[0, 0]], rows0, sem0).wait()
        pltpu.sync_copy(rows0, acc.at[dst_b.at[par, ci]], add=True)

        @pl.when(c0 + 2 < _CH)
        def _():
            cn = c0 + 2
            pltpu.async_copy(y_hbm.at[src_b.at[(cn // _BC) % 2, cn % _BC]],
                             rows0, sem0)

        pltpu.make_async_copy(y_hbm.at[src_b.at[0, 0]], rows1, sem1).wait()
        pltpu.sync_copy(rows1, acc.at[dst_b.at[par, ci + 1]], add=True)

        @pl.when(c0 + 3 < _CH)
        def _():
            cn = c0 + 3
            pltpu.async_copy(y_hbm.at[src_b.at[(cn // _BC) % 2, cn % _BC]],
                             rows1, sem1)

        return carry

    lax.fori_loop(0, _CH // 2, body, 0)
    plsc.subcore_barrier()
    pltpu.sync_copy(acc.at[pl.ds(row0, _RT)],
                    out_hbm.at[cid, pl.ds(row0, _RT)])


def _mm_first_body(x_ref, w_ref, o_ref):
    o_ref[...] = lax.dot_general(
        x_ref[...], w_ref[...], (((1,), (0,)), ((), ())),
        precision=lax.Precision.HIGHEST, preferred_element_type=jnp.float32)


def _mm_mid_body(p_ref, b_ref, w_ref, o_ref):
    h = jnp.maximum(p_ref[0] + p_ref[1] + b_ref[...], 0.0)
    o_ref[...] = lax.dot_general(
        h, w_ref[...], (((1,), (0,)), ((), ())),
        precision=lax.Precision.HIGHEST, preferred_element_type=jnp.float32)


def _relu_body(p_ref, b_ref, o_ref):
    o_ref[...] = jnp.maximum(p_ref[0] + p_ref[1] + b_ref[...], 0.0)


def _mm_first(x, w):
    return pl.pallas_call(
        _mm_first_body,
        grid=(_NP // _BLK,),
        in_specs=[
            pl.BlockSpec((_BLK, _D), lambda i: (i, 0)),
            pl.BlockSpec((_D, _D), lambda i: (0, 0)),
        ],
        out_specs=pl.BlockSpec((_BLK, _D), lambda i: (i, 0)),
        out_shape=jax.ShapeDtypeStruct((_NP, _D), jnp.float32),
    )(x, w)


def _mm_mid(p, b, w):
    return pl.pallas_call(
        _mm_mid_body,
        grid=(_NP // _BLK,),
        in_specs=[
            pl.BlockSpec((2, _BLK, _D), lambda i: (0, i, 0)),
            pl.BlockSpec((1, _D), lambda i: (0, 0)),
            pl.BlockSpec((_D, _D), lambda i: (0, 0)),
        ],
        out_specs=pl.BlockSpec((_BLK, _D), lambda i: (i, 0)),
        out_shape=jax.ShapeDtypeStruct((_NP, _D), jnp.float32),
    )(p, b.reshape(1, _D), w)


def _relu_out(p, b):
    return pl.pallas_call(
        _relu_body,
        grid=(_NP // _BLK,),
        in_specs=[
            pl.BlockSpec((2, _BLK, _D), lambda i: (0, i, 0)),
            pl.BlockSpec((1, _D), lambda i: (0, 0)),
        ],
        out_specs=pl.BlockSpec((_BLK, _D), lambda i: (i, 0)),
        out_shape=jax.ShapeDtypeStruct((_NP, _D), jnp.float32),
    )(p, b.reshape(1, _D))


def kernel(features, edge_index, W1, b1, W2, b2, W3, b3):
    f = jnp.pad(features, ((0, _NP - _N), (0, 0)))
    src = jnp.pad(edge_index[0],
                  (0, _EP - _E)).reshape(2, 16, _NBLK, _BC, _K)
    dst = jnp.pad(edge_index[1], (0, _EP - _E),
                  constant_values=_DUMMY).reshape(2, 16, _NBLK, _BC, _K)
    w3p = jnp.pad(W3, ((0, 0), (0, _D - _C)))
    b3p = jnp.pad(b3, (0, _D - _C))
    z = jnp.zeros((40, _D), jnp.float32)

    y1 = _mm_first(f, W1)
    p1 = _sc_propagate(y1, src, dst, z)
    y2 = _mm_mid(p1, b1, W2)
    p2 = _sc_propagate(y2, src, dst, z)
    y3 = _mm_mid(p2, b2, w3p)
    p3 = _sc_propagate(y3, src, dst, z)
    out = _relu_out(p3, b3p)
    return out[:_N, :_C]
